# Initial kernel scaffold; baseline (speedup 1.0000x reference)
import jax
import jax.numpy as jnp
from jax.experimental import pallas as pl


def kernel(zL, dictionary_enc, dictionary_dec, bias_pre, bias_enc):
    raise NotImplementedError("write your pallas kernel here")


# speedup vs baseline: 18.9023x; 18.9023x over previous
import functools

import jax
import jax.numpy as jnp
from jax.experimental import pallas as pl
from jax.experimental.pallas import tpu as pltpu

_TOPK = 64
_ROWS = 512


def _sae_block(x_ref, encT_ref, decT_ref, bpre_ref, benc_ref, zn_ref, xt_ref):
    xc = x_ref[...] - bpre_ref[...]
    logits = jnp.dot(xc, encT_ref[...], preferred_element_type=jnp.float32)
    z = jnp.maximum(logits + benc_ref[...], 0.0)
    zb = jax.lax.bitcast_convert_type(z, jnp.int32)
    t = jnp.zeros((z.shape[0], 1), jnp.int32)
    for b in range(30, -1, -1):
        cand = t | (1 << b)
        cnt = jnp.sum((zb >= cand).astype(jnp.int32), axis=1, keepdims=True)
        t = jnp.where(cnt >= _TOPK, cand, t)
    zs = jnp.where(zb >= t, z, 0.0)
    zn_ref[...] = zs
    xt_ref[...] = (
        jnp.dot(zs, decT_ref[...], preferred_element_type=jnp.float32)
        + bpre_ref[...]
    )


@functools.partial(jax.jit, static_argnames=())
def kernel(zL, dictionary_enc, dictionary_dec, bias_pre, bias_enc):
    B, D, L, H = zL.shape
    M = dictionary_enc.shape[0]
    N = B * D * L
    x = zL.reshape(N, H)
    encT = dictionary_enc.T
    decT = dictionary_dec.T
    bpre = bias_pre.reshape(1, H)
    benc = bias_enc.reshape(1, M)

    grid = (N // _ROWS,)
    zn_flat, xt_flat = pl.pallas_call(
        _sae_block,
        grid=grid,
        in_specs=[
            pl.BlockSpec((_ROWS, H), lambda i: (i, 0)),
            pl.BlockSpec((H, M), lambda i: (0, 0)),
            pl.BlockSpec((M, H), lambda i: (0, 0)),
            pl.BlockSpec((1, H), lambda i: (0, 0)),
            pl.BlockSpec((1, M), lambda i: (0, 0)),
        ],
        out_specs=[
            pl.BlockSpec((_ROWS, M), lambda i: (i, 0)),
            pl.BlockSpec((_ROWS, H), lambda i: (i, 0)),
        ],
        out_shape=[
            jax.ShapeDtypeStruct((N, M), jnp.float32),
            jax.ShapeDtypeStruct((N, H), jnp.float32),
        ],
        compiler_params=pltpu.CompilerParams(
            dimension_semantics=("arbitrary",),
        ),
    )(x, encT, decT, bpre, benc)
    return (zn_flat.reshape(B, D, L, M), xt_flat.reshape(B, D, L, H))
